# baseline (speedup 1.0000x reference)
import functools

import jax
import jax.numpy as jnp
from jax import lax
from jax.experimental import pallas as pl
from jax.experimental.pallas import tpu as pltpu
from jax.experimental.pallas import tpu_sc as plsc

_NW = 32
_RC = 16


def _bc16(x):
    return lax.broadcast_in_dim(x, (16,), ())


def _rsqrt16(x):
    i = lax.bitcast_convert_type(x, jnp.int32)
    i = jnp.int32(0x5F3759DF) - lax.shift_right_logical(i, 1)
    y = lax.bitcast_convert_type(i, jnp.float32)
    for _ in range(3):
        y = y * (jnp.float32(1.5) - jnp.float32(0.5) * x * y * y)
    return y


def _sc_body(hx_hbm, hy_hbm, hz_hbm, d_hbm, z_hbm, ox_hbm, oy_hbm, oz_hbm,
             d_v, hx_v, hy_v, hz_v, z_v, ox_v, oy_v, oz_v):
    B = 8192
    T = 64
    S = 64
    RW = B // _NW
    wid = lax.axis_index("s") * 2 + lax.axis_index("c")
    base = wid * RW

    def chunk_body(ci, carry):
        cbase = (base + ci * _RC) * T
        pltpu.sync_copy(d_hbm.at[pl.ds(cbase, _RC * T)], d_v)
        pltpu.sync_copy(hx_hbm.at[pl.ds(cbase, _RC * T)], hx_v)
        pltpu.sync_copy(hy_hbm.at[pl.ds(cbase, _RC * T)], hy_v)
        pltpu.sync_copy(hz_hbm.at[pl.ds(cbase, _RC * T)], hz_v)
        pltpu.sync_copy(z_hbm.at[pl.ds(cbase, _RC * S)], z_v)

        def ray_body(r, carry2):
            rb = r * T
            dg = [d_v[pl.ds(rb + 16 * g, 16)] for g in range(T // 16)]
            xg = [hx_v[pl.ds(rb + 16 * g, 16)] for g in range(T // 16)]
            yg = [hy_v[pl.ds(rb + 16 * g, 16)] for g in range(T // 16)]
            zg = [hz_v[pl.ds(rb + 16 * g, 16)] for g in range(T // 16)]
            lanes = [_bc16(jnp.int32(l)) for l in range(16)]

            def _take(v, il):
                return v.at[il].get(mode="promise_in_bounds",
                                    unique_indices=False)

            hx0 = _take(xg[0], lanes[0])
            hy0 = _take(yg[0], lanes[0])
            hz0 = _take(zg[0], lanes[0])
            for j in range(S // 16):
                z16 = z_v[pl.ds(r * S + 16 * j, 16)]
                b0 = jnp.full((16,), 10.0, jnp.float32)
                b1 = jnp.full((16,), 10.0, jnp.float32)
                px0, py0, pz0 = hx0, hy0, hz0
                px1, py1, pz1 = hx0, hy0, hz0
                for t in range(T):
                    g, l = divmod(t, 16)
                    dcol = _take(dg[g], lanes[l])
                    cx = _take(xg[g], lanes[l])
                    cy = _take(yg[g], lanes[l])
                    cz = _take(zg[g], lanes[l])
                    u = z16 - dcol
                    un = dcol - z16
                    c0 = (u >= 0.0) & (u < b0)
                    c1 = (un >= 0.0) & (un < b1)
                    b0 = jnp.where(c0, u, b0)
                    b1 = jnp.where(c1, un, b1)
                    px0 = jnp.where(c0, cx, px0)
                    py0 = jnp.where(c0, cy, py0)
                    pz0 = jnp.where(c0, cz, pz0)
                    px1 = jnp.where(c1, cx, px1)
                    py1 = jnp.where(c1, cy, py1)
                    pz1 = jnp.where(c1, cz, pz1)
                mx = (px1 - px0) / z16
                my = (py1 - py0) / z16
                mz = (pz1 - pz0) / z16
                nsq = mx * mx + my * my + mz * mz
                norm = nsq * _rsqrt16(nsq)
                ox_v[pl.ds(r * S + 16 * j, 16)] = px0 + b0 * (mx / norm)
                oy_v[pl.ds(r * S + 16 * j, 16)] = py0 + b0 * (my / norm)
                oz_v[pl.ds(r * S + 16 * j, 16)] = pz0 + b0 * (mz / norm)
            return carry2

        lax.fori_loop(0, _RC, ray_body, 0)
        pltpu.sync_copy(ox_v, ox_hbm.at[pl.ds(cbase, _RC * S)])
        pltpu.sync_copy(oy_v, oy_hbm.at[pl.ds(cbase, _RC * S)])
        pltpu.sync_copy(oz_v, oz_hbm.at[pl.ds(cbase, _RC * S)])
        return carry

    lax.fori_loop(0, RW // _RC, chunk_body, 0)


def kernel(r_hist, distances, z_vals):
    B, T = distances.shape
    S = z_vals.shape[1]
    hx = r_hist[..., 0].reshape(B * T)
    hy = r_hist[..., 1].reshape(B * T)
    hz = r_hist[..., 2].reshape(B * T)
    d = distances.reshape(B * T)
    z = z_vals[..., 0].reshape(B * S)
    mesh = plsc.VectorSubcoreMesh(core_axis_name="c", subcore_axis_name="s")
    k = functools.partial(
        pl.kernel,
        out_type=[jax.ShapeDtypeStruct((B * S,), jnp.float32)] * 3,
        mesh=mesh,
        scratch_types=[
            pltpu.VMEM((_RC * T,), jnp.float32),
            pltpu.VMEM((_RC * T,), jnp.float32),
            pltpu.VMEM((_RC * T,), jnp.float32),
            pltpu.VMEM((_RC * T,), jnp.float32),
            pltpu.VMEM((_RC * S,), jnp.float32),
            pltpu.VMEM((_RC * S,), jnp.float32),
            pltpu.VMEM((_RC * S,), jnp.float32),
            pltpu.VMEM((_RC * S,), jnp.float32),
        ],
    )(_sc_body)
    ox, oy, oz = k(hx, hy, hz, d, z)
    return jnp.stack([ox.reshape(B, S), oy.reshape(B, S),
                      oz.reshape(B, S)], axis=-1)


# speedup vs baseline: 2.8093x; 2.8093x over previous
import functools

import jax
import jax.numpy as jnp
from jax import lax
from jax.experimental import pallas as pl
from jax.experimental.pallas import tpu as pltpu
from jax.experimental.pallas import tpu_sc as plsc

_RBLK = 64
_NW = 32
_RC = 16
_BSC = 2560


def _bc16(x):
    return lax.broadcast_in_dim(x, (16,), ())


def _rsqrt16(x):
    i = lax.bitcast_convert_type(x, jnp.int32)
    i = jnp.int32(0x5F3759DF) - lax.shift_right_logical(i, 1)
    y = lax.bitcast_convert_type(i, jnp.float32)
    for _ in range(3):
        y = y * (jnp.float32(1.5) - jnp.float32(0.5) * x * y * y)
    return y


def _sc_body(hx_hbm, hy_hbm, hz_hbm, d_hbm, z_hbm, ox_hbm, oy_hbm, oz_hbm,
             d_v, hx_v, hy_v, hz_v, z_v, ox_v, oy_v, oz_v):
    T = 64
    S = 64
    RW = d_hbm.shape[0] // T // _NW
    wid = lax.axis_index("s") * 2 + lax.axis_index("c")
    base = wid * RW

    def chunk_body(ci, carry):
        cbase = (base + ci * _RC) * T
        pltpu.sync_copy(d_hbm.at[pl.ds(cbase, _RC * T)], d_v)
        pltpu.sync_copy(hx_hbm.at[pl.ds(cbase, _RC * T)], hx_v)
        pltpu.sync_copy(hy_hbm.at[pl.ds(cbase, _RC * T)], hy_v)
        pltpu.sync_copy(hz_hbm.at[pl.ds(cbase, _RC * T)], hz_v)
        pltpu.sync_copy(z_hbm.at[pl.ds(cbase, _RC * S)], z_v)

        def ray_body(r, carry2):
            rb = r * T
            dg = [d_v[pl.ds(rb + 16 * g, 16)] for g in range(T // 16)]
            xg = [hx_v[pl.ds(rb + 16 * g, 16)] for g in range(T // 16)]
            yg = [hy_v[pl.ds(rb + 16 * g, 16)] for g in range(T // 16)]
            zg = [hz_v[pl.ds(rb + 16 * g, 16)] for g in range(T // 16)]
            lanes = [_bc16(jnp.int32(l)) for l in range(16)]

            def _take(v, il):
                return v.at[il].get(mode="promise_in_bounds",
                                    unique_indices=False)

            hx0 = _take(xg[0], lanes[0])
            hy0 = _take(yg[0], lanes[0])
            hz0 = _take(zg[0], lanes[0])
            for j in range(S // 16):
                z16 = z_v[pl.ds(r * S + 16 * j, 16)]
                b0 = jnp.full((16,), 10.0, jnp.float32)
                b1 = jnp.full((16,), 10.0, jnp.float32)
                px0, py0, pz0 = hx0, hy0, hz0
                px1, py1, pz1 = hx0, hy0, hz0
                for t in range(T):
                    g, l = divmod(t, 16)
                    dcol = _take(dg[g], lanes[l])
                    cx = _take(xg[g], lanes[l])
                    cy = _take(yg[g], lanes[l])
                    cz = _take(zg[g], lanes[l])
                    u = z16 - dcol
                    un = dcol - z16
                    c0 = (u >= 0.0) & (u < b0)
                    c1 = (un >= 0.0) & (un < b1)
                    b0 = jnp.where(c0, u, b0)
                    b1 = jnp.where(c1, un, b1)
                    px0 = jnp.where(c0, cx, px0)
                    py0 = jnp.where(c0, cy, py0)
                    pz0 = jnp.where(c0, cz, pz0)
                    px1 = jnp.where(c1, cx, px1)
                    py1 = jnp.where(c1, cy, py1)
                    pz1 = jnp.where(c1, cz, pz1)
                mx = (px1 - px0) / z16
                my = (py1 - py0) / z16
                mz = (pz1 - pz0) / z16
                nsq = mx * mx + my * my + mz * mz
                norm = nsq * _rsqrt16(nsq)
                ox_v[pl.ds(r * S + 16 * j, 16)] = px0 + b0 * (mx / norm)
                oy_v[pl.ds(r * S + 16 * j, 16)] = py0 + b0 * (my / norm)
                oz_v[pl.ds(r * S + 16 * j, 16)] = pz0 + b0 * (mz / norm)
            return carry2

        lax.fori_loop(0, _RC, ray_body, 0)
        pltpu.sync_copy(ox_v, ox_hbm.at[pl.ds(cbase, _RC * S)])
        pltpu.sync_copy(oy_v, oy_hbm.at[pl.ds(cbase, _RC * S)])
        pltpu.sync_copy(oz_v, oz_hbm.at[pl.ds(cbase, _RC * S)])
        return carry

    lax.fori_loop(0, RW // _RC, chunk_body, 0)


def _sc_part(r_hist, distances, z):
    B, T = distances.shape
    S = z.shape[1]
    hx = r_hist[..., 0].reshape(B * T)
    hy = r_hist[..., 1].reshape(B * T)
    hz = r_hist[..., 2].reshape(B * T)
    d = distances.reshape(B * T)
    zf = z.reshape(B * S)
    mesh = plsc.VectorSubcoreMesh(core_axis_name="c", subcore_axis_name="s")
    k = functools.partial(
        pl.kernel,
        out_type=[jax.ShapeDtypeStruct((B * S,), jnp.float32)] * 3,
        mesh=mesh,
        scratch_types=[
            pltpu.VMEM((_RC * T,), jnp.float32),
            pltpu.VMEM((_RC * T,), jnp.float32),
            pltpu.VMEM((_RC * T,), jnp.float32),
            pltpu.VMEM((_RC * T,), jnp.float32),
            pltpu.VMEM((_RC * S,), jnp.float32),
            pltpu.VMEM((_RC * S,), jnp.float32),
            pltpu.VMEM((_RC * S,), jnp.float32),
            pltpu.VMEM((_RC * S,), jnp.float32),
        ],
    )(_sc_body)
    ox, oy, oz = k(hx, hy, hz, d, zf)
    return jnp.stack([ox.reshape(B, S), oy.reshape(B, S),
                      oz.reshape(B, S)], axis=-1)


def _evolve_block(hist_ref, d_ref, z_ref, out_ref):
    z = z_ref[...]
    d = d_ref[...]
    R, S = z.shape
    T = d.shape[1]
    L = 2 * S
    hx = hist_ref[0]
    hy = hist_ref[1]
    hz = hist_ref[2]
    zs = jnp.concatenate([z, -z], axis=-1)
    sgn = jnp.concatenate([jnp.full((R, S), -1.0, jnp.float32),
                           jnp.full((R, S), 1.0, jnp.float32)], axis=-1)
    best = jnp.full((R, L), 10.0, jnp.float32)
    px = jnp.broadcast_to(hx[:, 0:1], (R, L))
    py = jnp.broadcast_to(hy[:, 0:1], (R, L))
    pz = jnp.broadcast_to(hz[:, 0:1], (R, L))
    for t in range(T):
        dcol = jnp.broadcast_to(d[:, t:t + 1], (R, L))
        u = dcol * sgn + zs
        cond = (u >= 0.0) & (u < best)
        best = jnp.where(cond, u, best)
        px = jnp.where(cond, jnp.broadcast_to(hx[:, t:t + 1], (R, L)), px)
        py = jnp.where(cond, jnp.broadcast_to(hy[:, t:t + 1], (R, L)), py)
        pz = jnp.where(cond, jnp.broadcast_to(hz[:, t:t + 1], (R, L)), pz)
    vals = best[:, :S]
    mx = (px[:, S:] - px[:, :S]) / z
    my = (py[:, S:] - py[:, :S]) / z
    mz = (pz[:, S:] - pz[:, :S]) / z
    norm = jnp.sqrt(mx * mx + my * my + mz * mz)
    out_ref[0] = px[:, :S] + vals * (mx / norm)
    out_ref[1] = py[:, :S] + vals * (my / norm)
    out_ref[2] = pz[:, :S] + vals * (mz / norm)


def _tc_part(r_hist, distances, z):
    B, T = distances.shape
    S = z.shape[1]
    hist_t = jnp.transpose(r_hist, (2, 0, 1))
    out_t = pl.pallas_call(
        _evolve_block,
        grid=(B // _RBLK,),
        in_specs=[
            pl.BlockSpec((3, _RBLK, T), lambda i: (0, i, 0)),
            pl.BlockSpec((_RBLK, T), lambda i: (i, 0)),
            pl.BlockSpec((_RBLK, S), lambda i: (i, 0)),
        ],
        out_specs=pl.BlockSpec((3, _RBLK, S), lambda i: (0, i, 0)),
        out_shape=jax.ShapeDtypeStruct((3, B, S), jnp.float32),
    )(hist_t, distances, z)
    return jnp.transpose(out_t, (1, 2, 0))


def kernel(r_hist, distances, z_vals):
    z = z_vals[..., 0]
    out_sc = _sc_part(r_hist[:_BSC], distances[:_BSC], z[:_BSC])
    out_tc = _tc_part(r_hist[_BSC:], distances[_BSC:], z[_BSC:])
    return jnp.concatenate([out_sc, out_tc], axis=0)
